# Initial kernel scaffold; baseline (speedup 1.0000x reference)
import jax
import jax.numpy as jnp
from jax.experimental import pallas as pl


def kernel(x, edge_index, W_rel1, b_rel1, W_root1, W_rel2, b_rel2, W_root2, W_rel3, b_rel3, W_root3, W_lin, b_lin):
    raise NotImplementedError("write your pallas kernel here")


# speedup vs baseline: 9.1835x; 9.1835x over previous
import functools

import jax
import jax.numpy as jnp
from jax import lax
from jax.experimental import pallas as pl
from jax.experimental.pallas import tpu as pltpu
from jax.experimental.pallas import tpu_sc as plsc

N = 10000
D = 128
E = 320000
H = 20
L = 2

HP = 32
NC = 2
NS = 16
NW = NC * NS
CHUNK = 128
NCH = -(-E // (NW * CHUNK))
EPW = NCH * CHUNK
EPAD = EPW * NW
NP = 10112
RPT = NP // NS


def _tc_pre_body(x_ref, wr_ref, wo_ref, p_ref, r_ref):
    x = x_ref[...]
    p_ref[...] = jnp.dot(x, wr_ref[...], preferred_element_type=jnp.float32)
    r_ref[...] = jnp.dot(x, wo_ref[...], preferred_element_type=jnp.float32)


def _tc_pre(xp, wr, wo):
    return pl.pallas_call(
        _tc_pre_body,
        out_shape=[
            jax.ShapeDtypeStruct((NP, HP), jnp.float32),
            jax.ShapeDtypeStruct((NP, HP), jnp.float32),
        ],
    )(xp, wr, wo)


def _combine(agg_ref, r_ref, b_ref):
    t = agg_ref[0] + agg_ref[1] + r_ref[...] + b_ref[...]
    rows = lax.broadcasted_iota(jnp.int32, (NP, HP), 0)
    t = jnp.where(rows < N, t, 0.0)
    nrm = jnp.sqrt(jnp.sum(t * t, axis=1, keepdims=True))
    h = t / jnp.maximum(nrm, 1e-12)
    return jnp.maximum(h, 0.0)


def _tc_mid_body(agg_ref, r_ref, b_ref, wr_ref, wo_ref, p_ref, rn_ref):
    h = _combine(agg_ref, r_ref, b_ref)
    p_ref[...] = jnp.dot(h, wr_ref[...], preferred_element_type=jnp.float32)
    rn_ref[...] = jnp.dot(h, wo_ref[...], preferred_element_type=jnp.float32)


def _tc_mid(agg, r, b, wr, wo):
    return pl.pallas_call(
        _tc_mid_body,
        out_shape=[
            jax.ShapeDtypeStruct((NP, HP), jnp.float32),
            jax.ShapeDtypeStruct((NP, HP), jnp.float32),
        ],
    )(agg, r, b, wr, wo)


def _tc_fin_body(agg_ref, r_ref, b_ref, wm_ref, wn_ref, bl_ref, o_ref):
    h = _combine(agg_ref, r_ref, b_ref)
    max_p = jnp.max(h, axis=0, keepdims=True)
    mean_p = jnp.sum(h, axis=0, keepdims=True) * (1.0 / N)
    o_ref[...] = (
        jnp.dot(max_p, wm_ref[...], preferred_element_type=jnp.float32)
        + jnp.dot(mean_p, wn_ref[...], preferred_element_type=jnp.float32)
        + bl_ref[...]
    )


def _tc_fin(agg, r, b, wm, wn, bl):
    return pl.pallas_call(
        _tc_fin_body,
        out_shape=jax.ShapeDtypeStruct((1, L), jnp.float32),
    )(agg, r, b, wm, wn, bl)


def _sc_body(p_hbm, src_hbm, dst_hbm, zeros_hbm, out_hbm,
             idx_s, idx_d, rows, agg_sh, sem):
    c = lax.axis_index("c")
    s = lax.axis_index("s")
    w = c * NS + s
    pltpu.sync_copy(zeros_hbm.at[pl.ds(s * RPT, RPT)],
                    agg_sh.at[pl.ds(s * RPT, RPT)])
    pltpu.sync_copy(src_hbm.at[w], idx_s)
    pltpu.sync_copy(dst_hbm.at[w], idx_d)
    plsc.subcore_barrier()

    def step(j, carry):
        pltpu.async_copy(p_hbm.at[idx_s.at[j]], rows, sem).wait()
        pltpu.sync_copy(rows, agg_sh.at[idx_d.at[j]], add=True)
        return carry

    lax.fori_loop(0, NCH, step, 0)
    plsc.subcore_barrier()
    pltpu.sync_copy(agg_sh.at[pl.ds(s * RPT, RPT)],
                    out_hbm.at[c, pl.ds(s * RPT, RPT)])


_SC_MESH = plsc.VectorSubcoreMesh(
    core_axis_name="c", subcore_axis_name="s", num_cores=NC, num_subcores=NS)

_sc_scatter = pl.kernel(
    _sc_body,
    out_type=jax.ShapeDtypeStruct((NC, NP, HP), jnp.float32),
    mesh=_SC_MESH,
    scratch_types=[
        pltpu.VMEM((NCH, CHUNK), jnp.int32),
        pltpu.VMEM((NCH, CHUNK), jnp.int32),
        pltpu.VMEM((CHUNK, HP), jnp.float32),
        pltpu.VMEM_SHARED((NP, HP), jnp.float32),
        pltpu.SemaphoreType.DMA,
    ],
    compiler_params=pltpu.CompilerParams(use_tc_tiling_on_sc=False),
)


def _pad2(w, shape):
    out = jnp.zeros(shape, dtype=jnp.float32)
    return out.at[:w.shape[0], :w.shape[1]].set(w)


def kernel(x, edge_index, W_rel1, b_rel1, W_root1, W_rel2, b_rel2, W_root2,
           W_rel3, b_rel3, W_root3, W_lin, b_lin):
    xp = jnp.zeros((NP, D), jnp.float32).at[:N].set(x)
    pad = jnp.full((EPAD - E,), N, jnp.int32)
    src = jnp.concatenate([edge_index[0], pad]).reshape(NW, NCH, CHUNK)
    dst = jnp.concatenate([edge_index[1], pad]).reshape(NW, NCH, CHUNK)
    zeros = jnp.zeros((NP, HP), jnp.float32)

    wr1 = _pad2(W_rel1, (D, HP))
    wo1 = _pad2(W_root1, (D, HP))
    wr2 = _pad2(W_rel2, (HP, HP))
    wo2 = _pad2(W_root2, (HP, HP))
    wr3 = _pad2(W_rel3, (HP, HP))
    wo3 = _pad2(W_root3, (HP, HP))
    b1 = jnp.zeros((1, HP), jnp.float32).at[0, :H].set(b_rel1)
    b2 = jnp.zeros((1, HP), jnp.float32).at[0, :H].set(b_rel2)
    b3 = jnp.zeros((1, HP), jnp.float32).at[0, :H].set(b_rel3)
    wm = _pad2(W_lin[:H], (HP, L))
    wn = _pad2(W_lin[H:], (HP, L))
    bl = b_lin.reshape(1, L)

    p1, r1 = _tc_pre(xp, wr1, wo1)
    agg1 = _sc_scatter(p1, src, dst, zeros)
    p2, r2 = _tc_mid(agg1, r1, b1, wr2, wo2)
    agg2 = _sc_scatter(p2, src, dst, zeros)
    p3, r3 = _tc_mid(agg2, r2, b2, wr3, wo3)
    agg3 = _sc_scatter(p3, src, dst, zeros)
    return _tc_fin(agg3, r3, b3, wm, wn, bl)
